# baseline (speedup 1.0000x reference)
import functools

import jax
import jax.numpy as jnp
from jax import lax
from jax.experimental import pallas as pl
from jax.experimental.pallas import tpu as pltpu
from jax.experimental.pallas import tpu_sc as plsc

BATCH = 4096
FIELDS = 26
DIM = 64
NC = 2
NS = 16
NW = NC * NS
BW = BATCH // NW
DT = DIM // 8
NBUF = 13

_mesh = plsc.VectorSubcoreMesh(
    core_axis_name="c", subcore_axis_name="s", num_cores=NC, num_subcores=NS
)


@functools.partial(
    pl.kernel,
    out_type=jax.ShapeDtypeStruct((FIELDS, DT, NW, 8, BW), jnp.float32),
    mesh=_mesh,
    scratch_types=[
        pltpu.VMEM((FIELDS, BW), jnp.int32),
        pltpu.VMEM((NBUF, BW, DIM), jnp.float32),
        pltpu.VMEM((2, DIM, BW + 1), jnp.float32),
        pltpu.SemaphoreType.DMA((NBUF,)),
        pltpu.SemaphoreType.DMA((2,)),
    ],
    compiler_params=pltpu.CompilerParams(
        use_tc_tiling_on_sc=False, needs_layout_passes=False
    ),
)
def _sc_gather(idx_hbm, table_hbm, out_hbm, idx_v, rows_v, rt_v, gsem, ssem):
    wid = lax.axis_index("s") * NC + lax.axis_index("c")
    pltpu.sync_copy(idx_hbm.at[wid], idx_v)
    iota16 = lax.iota(jnp.int32, 16)
    dvecs = [iota16 + d0 * 16 for d0 in range(DIM // 16)]

    def gather(f, side):
        return pltpu.async_copy(
            table_hbm.at[idx_v.at[f]], rows_v.at[side], gsem.at[side]
        )

    def wait_gather(side):
        pltpu.make_async_copy(
            table_hbm.at[idx_v.at[0]], rows_v.at[side], gsem.at[side]
        ).wait()

    def fire_stores(f, side):
        for dt in range(DT):
            pltpu.async_copy(
                rt_v.at[side, pl.ds(dt * 8, 8), pl.ds(0, BW)],
                out_hbm.at[f, dt, wid],
                ssem.at[side],
            )

    def wait_stores(side):
        for dt in range(DT):
            pltpu.make_async_copy(
                rt_v.at[side, pl.ds(dt * 8, 8), pl.ds(0, BW)],
                out_hbm.at[0, dt, wid],
                ssem.at[side],
            ).wait()

    def transpose(b_in, side):
        rows = rows_v.at[b_in]
        rt = rt_v.at[side]

        @plsc.parallel_loop(0, BW, unroll=8)
        def tb(b):
            for d0 in range(DIM // 16):
                v = rows[b, pl.ds(d0 * 16, 16)]
                plsc.store_scatter(rt, [dvecs[d0], jnp.full((16,), b, jnp.int32)], v)

    for b in range(NBUF):
        gather(b, b)
    for j in range(FIELDS):
        b = j % NBUF
        side = j % 2
        wait_gather(b)

        if j >= 2:
            wait_stores(side)
        transpose(b, side)
        fire_stores(j, side)

        if j + NBUF < FIELDS:
            gather(j + NBUF, b)
    wait_stores(0)
    wait_stores(1)


def kernel(x, table):
    idx = x.astype(jnp.int32).reshape(NW, BW, FIELDS).transpose(0, 2, 1)
    out = _sc_gather(idx, table)
    out = out.reshape(FIELDS, DT, NW, 8, BW)
    return out.transpose(2, 4, 0, 1, 3).reshape(BATCH, FIELDS, DIM)


# speedup vs baseline: 1.1138x; 1.1138x over previous
import functools

import jax
import jax.numpy as jnp
from jax import lax
from jax.experimental import pallas as pl
from jax.experimental.pallas import tpu as pltpu
from jax.experimental.pallas import tpu_sc as plsc

BATCH = 4096
FIELDS = 26
DIM = 64
NC = 2
NS = 16
NW = NC * NS
BW = BATCH // NW
DT = DIM // 8
NBUF = 13

_mesh = plsc.VectorSubcoreMesh(
    core_axis_name="c", subcore_axis_name="s", num_cores=NC, num_subcores=NS
)


@functools.partial(
    pl.kernel,
    out_type=jax.ShapeDtypeStruct((FIELDS, DT, NW, 8, BW), jnp.float32),
    mesh=_mesh,
    scratch_types=[
        pltpu.VMEM((FIELDS, BW), jnp.int32),
        pltpu.VMEM((NBUF, BW, DIM), jnp.float32),
        pltpu.VMEM((2, DIM, BW + 1), jnp.float32),
        pltpu.SemaphoreType.DMA((NBUF,)),
        pltpu.SemaphoreType.DMA((2,)),
    ],
    compiler_params=pltpu.CompilerParams(
        use_tc_tiling_on_sc=False, needs_layout_passes=False
    ),
)
def _sc_gather(idx_hbm, table_hbm, out_hbm, idx_v, rows_v, rt_v, gsem, ssem):
    wid = lax.axis_index("s") * NC + lax.axis_index("c")
    pltpu.sync_copy(idx_hbm.at[wid], idx_v)
    iota16 = lax.iota(jnp.int32, 16)
    dvecs = [iota16 + d0 * 16 for d0 in range(DIM // 16)]

    def gather(f, side):
        return pltpu.async_copy(
            table_hbm.at[idx_v.at[f]], rows_v.at[side], gsem.at[side]
        )

    def wait_gather(side):
        pltpu.make_async_copy(
            table_hbm.at[idx_v.at[0]], rows_v.at[side], gsem.at[side]
        ).wait()

    def fire_stores(f, side):
        for dt in range(DT):
            pltpu.async_copy(
                rt_v.at[side, pl.ds(dt * 8, 8), pl.ds(0, BW)],
                out_hbm.at[f, dt, wid],
                ssem.at[side],
            )

    def wait_stores(side):
        for dt in range(DT):
            pltpu.make_async_copy(
                rt_v.at[side, pl.ds(dt * 8, 8), pl.ds(0, BW)],
                out_hbm.at[0, dt, wid],
                ssem.at[side],
            ).wait()

    def transpose(b_in, side):
        rows = rows_v.at[b_in]
        rt = rt_v.at[side]

        @plsc.parallel_loop(0, BW, unroll=8)
        def tb(b):
            for d0 in range(DIM // 16):
                v = rows[b, pl.ds(d0 * 16, 16)]
                plsc.store_scatter(rt, [dvecs[d0], jnp.full((16,), b, jnp.int32)], v)

    for b in range(NBUF):
        gather(b, b)

    def body(j, carry):
        b = j % NBUF
        side = j % 2
        wait_gather(b)

        @pl.when(j >= 2)
        def _():
            wait_stores(side)

        transpose(b, side)
        fire_stores(j, side)

        @pl.when(j + NBUF < FIELDS)
        def _():
            gather(j + NBUF, b)

        return carry

    lax.fori_loop(0, FIELDS, body, 0)
    wait_stores(0)
    wait_stores(1)


def kernel(x, table):
    idx = x.astype(jnp.int32).reshape(NW, BW, FIELDS).transpose(0, 2, 1)
    out = _sc_gather(idx, table)
    out = out.reshape(FIELDS, DT, NW, 8, BW)
    return out.transpose(2, 4, 0, 1, 3).reshape(BATCH, FIELDS, DIM)
